# baseline (speedup 1.0000x reference)
import functools

import jax
import jax.numpy as jnp
from jax import lax
from jax.experimental import pallas as pl
from jax.experimental.pallas import tpu as pltpu
from jax.experimental.pallas import tpu_sc as plsc

N_NODES = 10000
D_IN = 128
D_HID = 256
N_CLS = 4

NC = 2
NS = 16
NW = NC * NS
CH = 64
NBUF = 4
IDXB = 32
LANES = 16
NPAD = 10240


def _fill1d(ref, value):
    v = jnp.full((LANES,), value, jnp.float32)
    for j in range(CH // LANES):
        ref[pl.ds(j * LANES, LANES)] = v


def _zero2d(ref):
    z = jnp.zeros((LANES,), jnp.float32)

    def body(r, _):
        for j in range(D_IN // LANES):
            ref[r, pl.ds(j * LANES, LANES)] = z
        return 0

    lax.fori_loop(0, CH, body, 0)


def _deg_body(n_groups, dst_hbm, out_hbm, dst_v, ones_v, zero_v, acc):
    cid = lax.axis_index("c")
    sid = lax.axis_index("s")
    wid = sid * NC + cid
    pltpu.sync_copy(dst_hbm.at[wid], dst_v)
    _fill1d(ones_v, 1.0)
    _fill1d(zero_v, 0.0)
    words = NPAD // NS
    base = sid * words
    for j in range(words // CH):
        pltpu.sync_copy(zero_v, acc.at[pl.ds(base + j * CH, CH)])
    plsc.subcore_barrier()

    def body(k, _):
        pltpu.sync_copy(ones_v, acc.at[dst_v.at[k // IDXB, k % IDXB]], add=True)
        return 0

    lax.fori_loop(0, n_groups * IDXB, body, 0)
    plsc.subcore_barrier()
    pltpu.sync_copy(acc.at[pl.ds(base, words)], out_hbm.at[cid, pl.ds(base, words)])


@functools.lru_cache(maxsize=None)
def _make_deg(n_groups):
    mesh = plsc.VectorSubcoreMesh(core_axis_name="c", subcore_axis_name="s")
    return pl.kernel(
        functools.partial(_deg_body, n_groups),
        out_type=jax.ShapeDtypeStruct((NC, NPAD), jnp.float32),
        mesh=mesh,
        scratch_types=[
            pltpu.VMEM((n_groups, IDXB, CH), jnp.int32),
            pltpu.VMEM((CH,), jnp.float32),
            pltpu.VMEM((CH,), jnp.float32),
            pltpu.VMEM_SHARED((NPAD,), jnp.float32),
        ],
    )


def _prop_body(n_groups, x_hbm, src_hbm, dst_hbm, out_hbm,
               si, di, rows_v, acc, *sems_all):
    gsems = sems_all[:NBUF]
    isem_s, isem_d = sems_all[NBUF:]
    cid = lax.axis_index("c")
    sid = lax.axis_index("s")
    wid = sid * NC + cid
    pltpu.async_copy(src_hbm.at[wid, 0], si.at[0], isem_s)
    pltpu.async_copy(dst_hbm.at[wid, 0], di.at[0], isem_d)
    _zero2d(rows_v.at[0])
    rows_per_tile = NPAD // NS
    base = sid * rows_per_tile
    for j in range(rows_per_tile // CH):
        pltpu.sync_copy(rows_v.at[0], acc.at[pl.ds(base + j * CH, CH)])
    plsc.subcore_barrier()
    pltpu.make_async_copy(src_hbm.at[wid, 0], si.at[0], isem_s).wait()
    pltpu.make_async_copy(dst_hbm.at[wid, 0], di.at[0], isem_d).wait()

    for g in range(n_groups):
        s = g % 2

        for b in range(NBUF):
            pltpu.async_copy(x_hbm.at[si.at[s, b]], rows_v.at[b], gsems[b])
        if g + 1 < n_groups:
            pltpu.async_copy(src_hbm.at[wid, g + 1], si.at[1 - s], isem_s)
            pltpu.async_copy(dst_hbm.at[wid, g + 1], di.at[1 - s], isem_d)

        def step(k, b, s=s):
            pltpu.make_async_copy(x_hbm.at[si.at[s, k]], rows_v.at[b], gsems[b]).wait()
            pltpu.sync_copy(rows_v.at[b], acc.at[di.at[s, k]], add=True)

        def body(it, _, s=s):
            kk = it * NBUF
            for b in range(NBUF):
                step(kk + b, b)
                pltpu.async_copy(x_hbm.at[si.at[s, kk + b + NBUF]], rows_v.at[b],
                                 gsems[b])
            return 0

        lax.fori_loop(0, (IDXB - NBUF) // NBUF, body, 0)
        for b in range(NBUF):
            step(IDXB - NBUF + b, b)
        if g + 1 < n_groups:
            pltpu.make_async_copy(src_hbm.at[wid, g + 1], si.at[1 - s], isem_s).wait()
            pltpu.make_async_copy(dst_hbm.at[wid, g + 1], di.at[1 - s], isem_d).wait()
    plsc.subcore_barrier()
    pltpu.sync_copy(acc.at[pl.ds(base, rows_per_tile)],
                    out_hbm.at[cid, pl.ds(base, rows_per_tile)])


@functools.lru_cache(maxsize=None)
def _make_prop(n_groups):
    mesh = plsc.VectorSubcoreMesh(core_axis_name="c", subcore_axis_name="s")
    return pl.kernel(
        functools.partial(_prop_body, n_groups),
        out_type=jax.ShapeDtypeStruct((NC, NPAD, D_IN), jnp.float32),
        mesh=mesh,
        scratch_types=[
            pltpu.VMEM((2, IDXB, CH), jnp.int32),
            pltpu.VMEM((2, IDXB, CH), jnp.int32),
            pltpu.VMEM((NBUF, CH, D_IN), jnp.float32),
            pltpu.VMEM_SHARED((NPAD, D_IN), jnp.float32),
        ] + [pltpu.SemaphoreType.DMA] * (NBUF + 2),
    )


BR = 2048
GRID = NPAD // BR


def _scale_body(deg_ref, x_ref, isd_ref, xs_ref):
    d = deg_ref[...]
    s = lax.rsqrt(jnp.maximum(d[:, 0:1] + d[:, 1:2], 1.0))
    isd_ref[...] = s
    xs_ref[...] = x_ref[...] * s


_scale = pl.pallas_call(
    _scale_body,
    grid=(GRID,),
    in_specs=[pl.BlockSpec((BR, NC), lambda i: (i, 0)),
              pl.BlockSpec((BR, D_IN), lambda i: (i, 0))],
    out_specs=[pl.BlockSpec((BR, 1), lambda i: (i, 0)),
               pl.BlockSpec((BR, D_IN), lambda i: (i, 0))],
    out_shape=[jax.ShapeDtypeStruct((NPAD, 1), jnp.float32),
               jax.ShapeDtypeStruct((NPAD, D_IN), jnp.float32)],
)


def _mid_body(p_ref, isd_ref, w1_ref, b1_ref, w2_ref, out_ref):
    p = (p_ref[0] + p_ref[1]) * isd_ref[...]
    h1 = jnp.dot(p, w1_ref[...], preferred_element_type=jnp.float32)
    h1 = jnp.maximum(h1 + b1_ref[...], 0.0)
    g = jnp.dot(h1, w2_ref[...], preferred_element_type=jnp.float32)
    out_ref[...] = g * isd_ref[...]


_mid = pl.pallas_call(
    _mid_body,
    grid=(GRID,),
    in_specs=[pl.BlockSpec((NC, BR, D_IN), lambda i: (0, i, 0)),
              pl.BlockSpec((BR, 1), lambda i: (i, 0)),
              pl.BlockSpec((D_IN, D_HID), lambda i: (0, 0)),
              pl.BlockSpec((1, D_HID), lambda i: (0, 0)),
              pl.BlockSpec((D_HID, D_IN), lambda i: (0, 0))],
    out_specs=pl.BlockSpec((BR, D_IN), lambda i: (i, 0)),
    out_shape=jax.ShapeDtypeStruct((NPAD, D_IN), jnp.float32),
)


def _fin_body(q_ref, isd_ref, b2_ref, wc_ref, bc_ref, out_ref, acc_ref):
    i = pl.program_id(0)

    @pl.when(i == 0)
    def _():
        acc_ref[...] = jnp.zeros_like(acc_ref)

    q = (q_ref[0] + q_ref[1]) * isd_ref[...]
    h2 = jnp.maximum(q + b2_ref[...], 0.0)
    rows = i * BR + lax.broadcasted_iota(jnp.int32, (BR, 1), 0)
    h2 = jnp.where(rows < N_NODES, h2, 0.0)
    acc_ref[...] += jnp.sum(h2, axis=0, keepdims=True)

    @pl.when(i == GRID - 1)
    def _():
        g = acc_ref[...] * (1.0 / N_NODES)
        out_ref[...] = jnp.dot(g, wc_ref[...],
                               preferred_element_type=jnp.float32) + bc_ref[...]


_fin = pl.pallas_call(
    _fin_body,
    grid=(GRID,),
    in_specs=[pl.BlockSpec((NC, BR, D_IN), lambda i: (0, i, 0)),
              pl.BlockSpec((BR, 1), lambda i: (i, 0)),
              pl.BlockSpec((1, D_IN), lambda i: (0, 0)),
              pl.BlockSpec((D_IN, N_CLS), lambda i: (0, 0)),
              pl.BlockSpec((1, N_CLS), lambda i: (0, 0))],
    out_specs=pl.BlockSpec((1, N_CLS), lambda i: (0, 0)),
    out_shape=jax.ShapeDtypeStruct((1, N_CLS), jnp.float32),
    scratch_shapes=[pltpu.VMEM((1, D_IN), jnp.float32)],
)


def kernel(x, edge_index, W1, b1, W2, b2, Wc, bc):
    src = edge_index[0]
    dst = edge_index[1]
    e = src.shape[0]
    n_groups = -(-e // (NW * CH * IDXB))
    e_pad = NW * CH * IDXB * n_groups
    fill = jnp.full((e_pad - e,), N_NODES, jnp.int32)
    srcr = (jnp.concatenate([src, fill]) % 640).reshape(NW, n_groups, IDXB, CH)
    dstr = jnp.concatenate([dst, fill]).reshape(NW, n_groups, IDXB, CH)
    x_pad = jnp.zeros((NPAD, D_IN), jnp.float32).at[:N_NODES].set(x)

    deg2 = _make_deg(n_groups)(dstr)
    isd, xs = _scale(deg2.T, x_pad)
    p = _make_prop(n_groups)(xs, srcr, dstr)
    g = _mid(p, isd, W1, b1.reshape(1, -1), W2)
    q = _make_prop(n_groups)(g, srcr, dstr)
    logits = _fin(q, isd, b2.reshape(1, -1), Wc, bc.reshape(1, -1))
    return logits.reshape(N_CLS)


# speedup vs baseline: 2.9268x; 2.9268x over previous
import functools

import jax
import jax.numpy as jnp
from jax import lax
from jax.experimental import pallas as pl
from jax.experimental.pallas import tpu as pltpu
from jax.experimental.pallas import tpu_sc as plsc

N_NODES = 10000
D_IN = 128
D_HID = 256
N_CLS = 4

NC = 2
NS = 16
NW = NC * NS
CH = 128
NBUF = 2
IDXB = 16
LANES = 16
NPAD = 10240


def _fill1d(ref, value):
    v = jnp.full((LANES,), value, jnp.float32)
    for j in range(CH // LANES):
        ref[pl.ds(j * LANES, LANES)] = v


def _zero2d(ref):
    z = jnp.zeros((LANES,), jnp.float32)

    def body(r, _):
        for j in range(D_IN // LANES):
            ref[r, pl.ds(j * LANES, LANES)] = z
        return 0

    lax.fori_loop(0, CH, body, 0)


def _deg_body(n_groups, dst_hbm, out_hbm, dst_v, ones_v, zero_v, acc):
    cid = lax.axis_index("c")
    sid = lax.axis_index("s")
    wid = sid * NC + cid
    pltpu.sync_copy(dst_hbm.at[wid], dst_v)
    _fill1d(ones_v, 1.0)
    _fill1d(zero_v, 0.0)
    words = NPAD // NS
    base = sid * words
    for j in range(words // CH):
        pltpu.sync_copy(zero_v, acc.at[pl.ds(base + j * CH, CH)])
    plsc.subcore_barrier()

    def body(k, _):
        pltpu.sync_copy(ones_v, acc.at[dst_v.at[k // IDXB, k % IDXB]], add=True)
        return 0

    lax.fori_loop(0, n_groups * IDXB, body, 0)
    plsc.subcore_barrier()
    pltpu.sync_copy(acc.at[pl.ds(base, words)], out_hbm.at[cid, pl.ds(base, words)])


@functools.lru_cache(maxsize=None)
def _make_deg(n_groups):
    mesh = plsc.VectorSubcoreMesh(core_axis_name="c", subcore_axis_name="s")
    return pl.kernel(
        functools.partial(_deg_body, n_groups),
        out_type=jax.ShapeDtypeStruct((NC, NPAD), jnp.float32),
        mesh=mesh,
        scratch_types=[
            pltpu.VMEM((n_groups, IDXB, CH), jnp.int32),
            pltpu.VMEM((CH,), jnp.float32),
            pltpu.VMEM((CH,), jnp.float32),
            pltpu.VMEM_SHARED((NPAD,), jnp.float32),
        ],
    )


def _prop_body(n_groups, x_hbm, src_hbm, dst_hbm, out_hbm,
               si, di, rows_v, acc, *sems_all):
    gsems = sems_all[:NBUF]
    isem_s, isem_d = sems_all[NBUF:]
    cid = lax.axis_index("c")
    sid = lax.axis_index("s")
    wid = sid * NC + cid
    pltpu.async_copy(src_hbm.at[wid, 0], si.at[0], isem_s)
    pltpu.async_copy(dst_hbm.at[wid, 0], di.at[0], isem_d)
    _zero2d(rows_v.at[0])
    rows_per_tile = NPAD // NS
    base = sid * rows_per_tile
    for j in range(rows_per_tile // CH):
        pltpu.sync_copy(rows_v.at[0], acc.at[pl.ds(base + j * CH, CH)])
    plsc.subcore_barrier()
    pltpu.make_async_copy(src_hbm.at[wid, 0], si.at[0], isem_s).wait()
    pltpu.make_async_copy(dst_hbm.at[wid, 0], di.at[0], isem_d).wait()

    for g in range(n_groups):
        s = g % 2

        for b in range(NBUF):
            pltpu.async_copy(x_hbm.at[si.at[s, b]], rows_v.at[b], gsems[b])
        if g + 1 < n_groups:
            pltpu.async_copy(src_hbm.at[wid, g + 1], si.at[1 - s], isem_s)
            pltpu.async_copy(dst_hbm.at[wid, g + 1], di.at[1 - s], isem_d)

        def step(k, b, s=s):
            pltpu.make_async_copy(x_hbm.at[si.at[s, k]], rows_v.at[b], gsems[b]).wait()
            pltpu.sync_copy(rows_v.at[b], acc.at[di.at[s, k]], add=True)

        def body(it, _, s=s):
            kk = it * NBUF
            for b in range(NBUF):
                step(kk + b, b)
                pltpu.async_copy(x_hbm.at[si.at[s, kk + b + NBUF]], rows_v.at[b],
                                 gsems[b])
            return 0

        lax.fori_loop(0, (IDXB - NBUF) // NBUF, body, 0)
        for b in range(NBUF):
            step(IDXB - NBUF + b, b)
        if g + 1 < n_groups:
            pltpu.make_async_copy(src_hbm.at[wid, g + 1], si.at[1 - s], isem_s).wait()
            pltpu.make_async_copy(dst_hbm.at[wid, g + 1], di.at[1 - s], isem_d).wait()
    plsc.subcore_barrier()
    pltpu.sync_copy(acc.at[pl.ds(base, rows_per_tile)],
                    out_hbm.at[cid, pl.ds(base, rows_per_tile)])


@functools.lru_cache(maxsize=None)
def _make_prop(n_groups):
    mesh = plsc.VectorSubcoreMesh(core_axis_name="c", subcore_axis_name="s")
    return pl.kernel(
        functools.partial(_prop_body, n_groups),
        out_type=jax.ShapeDtypeStruct((NC, NPAD, D_IN), jnp.float32),
        mesh=mesh,
        scratch_types=[
            pltpu.VMEM((2, IDXB, CH), jnp.int32),
            pltpu.VMEM((2, IDXB, CH), jnp.int32),
            pltpu.VMEM((NBUF, CH, D_IN), jnp.float32),
            pltpu.VMEM_SHARED((NPAD, D_IN), jnp.float32),
        ] + [pltpu.SemaphoreType.DMA] * (NBUF + 2),
    )


BR = 2048
GRID = NPAD // BR


def _scale_body(deg_ref, x_ref, isd_ref, xs_ref):
    d = deg_ref[...]
    s = lax.rsqrt(jnp.maximum(d[:, 0:1] + d[:, 1:2], 1.0))
    isd_ref[...] = s
    xs_ref[...] = x_ref[...] * s


_scale = pl.pallas_call(
    _scale_body,
    grid=(GRID,),
    in_specs=[pl.BlockSpec((BR, NC), lambda i: (i, 0)),
              pl.BlockSpec((BR, D_IN), lambda i: (i, 0))],
    out_specs=[pl.BlockSpec((BR, 1), lambda i: (i, 0)),
               pl.BlockSpec((BR, D_IN), lambda i: (i, 0))],
    out_shape=[jax.ShapeDtypeStruct((NPAD, 1), jnp.float32),
               jax.ShapeDtypeStruct((NPAD, D_IN), jnp.float32)],
)


def _mid_body(p_ref, isd_ref, w1_ref, b1_ref, w2_ref, out_ref):
    p = (p_ref[0] + p_ref[1]) * isd_ref[...]
    h1 = jnp.dot(p, w1_ref[...], preferred_element_type=jnp.float32)
    h1 = jnp.maximum(h1 + b1_ref[...], 0.0)
    g = jnp.dot(h1, w2_ref[...], preferred_element_type=jnp.float32)
    out_ref[...] = g * isd_ref[...]


_mid = pl.pallas_call(
    _mid_body,
    grid=(GRID,),
    in_specs=[pl.BlockSpec((NC, BR, D_IN), lambda i: (0, i, 0)),
              pl.BlockSpec((BR, 1), lambda i: (i, 0)),
              pl.BlockSpec((D_IN, D_HID), lambda i: (0, 0)),
              pl.BlockSpec((1, D_HID), lambda i: (0, 0)),
              pl.BlockSpec((D_HID, D_IN), lambda i: (0, 0))],
    out_specs=pl.BlockSpec((BR, D_IN), lambda i: (i, 0)),
    out_shape=jax.ShapeDtypeStruct((NPAD, D_IN), jnp.float32),
)


def _fin_body(q_ref, isd_ref, b2_ref, wc_ref, bc_ref, out_ref, acc_ref):
    i = pl.program_id(0)

    @pl.when(i == 0)
    def _():
        acc_ref[...] = jnp.zeros_like(acc_ref)

    q = (q_ref[0] + q_ref[1]) * isd_ref[...]
    h2 = jnp.maximum(q + b2_ref[...], 0.0)
    rows = i * BR + lax.broadcasted_iota(jnp.int32, (BR, 1), 0)
    h2 = jnp.where(rows < N_NODES, h2, 0.0)
    acc_ref[...] += jnp.sum(h2, axis=0, keepdims=True)

    @pl.when(i == GRID - 1)
    def _():
        g = acc_ref[...] * (1.0 / N_NODES)
        out_ref[...] = jnp.dot(g, wc_ref[...],
                               preferred_element_type=jnp.float32) + bc_ref[...]


_fin = pl.pallas_call(
    _fin_body,
    grid=(GRID,),
    in_specs=[pl.BlockSpec((NC, BR, D_IN), lambda i: (0, i, 0)),
              pl.BlockSpec((BR, 1), lambda i: (i, 0)),
              pl.BlockSpec((1, D_IN), lambda i: (0, 0)),
              pl.BlockSpec((D_IN, N_CLS), lambda i: (0, 0)),
              pl.BlockSpec((1, N_CLS), lambda i: (0, 0))],
    out_specs=pl.BlockSpec((1, N_CLS), lambda i: (0, 0)),
    out_shape=jax.ShapeDtypeStruct((1, N_CLS), jnp.float32),
    scratch_shapes=[pltpu.VMEM((1, D_IN), jnp.float32)],
)


def kernel(x, edge_index, W1, b1, W2, b2, Wc, bc):
    src = edge_index[0]
    dst = edge_index[1]
    e = src.shape[0]
    n_groups = -(-e // (NW * CH * IDXB))
    e_pad = NW * CH * IDXB * n_groups
    fill = N_NODES + (jnp.arange(e_pad - e, dtype=jnp.int32) % (NPAD - N_NODES))
    srcr = jnp.concatenate([src, fill]).reshape(NW, n_groups, IDXB, CH)
    dstr = jnp.concatenate([dst, fill]).reshape(NW, n_groups, IDXB, CH)
    x_pad = jnp.zeros((NPAD, D_IN), jnp.float32).at[:N_NODES].set(x)

    deg2 = _make_deg(n_groups)(dstr)
    isd, xs = _scale(deg2.T, x_pad)
    p = _make_prop(n_groups)(xs, srcr, dstr)
    g = _mid(p, isd, W1, b1.reshape(1, -1), W2)
    q = _make_prop(n_groups)(g, srcr, dstr)
    logits = _fin(q, isd, b2.reshape(1, -1), Wc, bc.reshape(1, -1))
    return logits.reshape(N_CLS)


# speedup vs baseline: 3.1375x; 1.0720x over previous
import functools

import jax
import jax.numpy as jnp
from jax import lax
from jax.experimental import pallas as pl
from jax.experimental.pallas import tpu as pltpu
from jax.experimental.pallas import tpu_sc as plsc

N_NODES = 10000
D_IN = 128
D_HID = 256
N_CLS = 4

NC = 2
NS = 16
NW = NC * NS
CH = 64
NBUF = 4
IDXB = 32
LANES = 16
NPAD = 10240


def _fill1d(ref, value):
    v = jnp.full((LANES,), value, jnp.float32)
    for j in range(CH // LANES):
        ref[pl.ds(j * LANES, LANES)] = v


def _zero2d(ref):
    z = jnp.zeros((LANES,), jnp.float32)

    def body(r, _):
        for j in range(D_IN // LANES):
            ref[r, pl.ds(j * LANES, LANES)] = z
        return 0

    lax.fori_loop(0, CH, body, 0)


def _deg_body(n_groups, dst_hbm, out_hbm, dst_v, ones_v, zero_v, acc):
    cid = lax.axis_index("c")
    sid = lax.axis_index("s")
    wid = sid * NC + cid
    pltpu.sync_copy(dst_hbm.at[wid], dst_v)
    _fill1d(ones_v, 1.0)
    _fill1d(zero_v, 0.0)
    words = NPAD // NS
    base = sid * words
    for j in range(words // CH):
        pltpu.sync_copy(zero_v, acc.at[pl.ds(base + j * CH, CH)])
    plsc.subcore_barrier()

    def body(k, _):
        pltpu.sync_copy(ones_v, acc.at[dst_v.at[k // IDXB, k % IDXB]], add=True)
        return 0

    lax.fori_loop(0, n_groups * IDXB, body, 0)
    plsc.subcore_barrier()
    pltpu.sync_copy(acc.at[pl.ds(base, words)], out_hbm.at[cid, pl.ds(base, words)])


@functools.lru_cache(maxsize=None)
def _make_deg(n_groups):
    mesh = plsc.VectorSubcoreMesh(core_axis_name="c", subcore_axis_name="s")
    return pl.kernel(
        functools.partial(_deg_body, n_groups),
        out_type=jax.ShapeDtypeStruct((NC, NPAD), jnp.float32),
        mesh=mesh,
        scratch_types=[
            pltpu.VMEM((n_groups, IDXB, CH), jnp.int32),
            pltpu.VMEM((CH,), jnp.float32),
            pltpu.VMEM((CH,), jnp.float32),
            pltpu.VMEM_SHARED((NPAD,), jnp.float32),
        ],
    )


def _prop_body(n_groups, x_hbm, src_hbm, dst_hbm, out_hbm,
               si, di, rows_v, acc, *sems_all):
    gsems = sems_all[:NBUF]
    isem_s, isem_d = sems_all[NBUF:]
    cid = lax.axis_index("c")
    sid = lax.axis_index("s")
    wid = sid * NC + cid
    pltpu.async_copy(src_hbm.at[wid, 0], si.at[0], isem_s)
    pltpu.async_copy(dst_hbm.at[wid, 0], di.at[0], isem_d)
    _zero2d(rows_v.at[0])
    rows_per_tile = NPAD // NS
    base = sid * rows_per_tile
    for j in range(rows_per_tile // CH):
        pltpu.sync_copy(rows_v.at[0], acc.at[pl.ds(base + j * CH, CH)])
    plsc.subcore_barrier()
    pltpu.make_async_copy(src_hbm.at[wid, 0], si.at[0], isem_s).wait()
    pltpu.make_async_copy(dst_hbm.at[wid, 0], di.at[0], isem_d).wait()

    for g in range(n_groups):
        s = g % 2

        for b in range(NBUF):
            pltpu.async_copy(x_hbm.at[si.at[s, b]], rows_v.at[b], gsems[b])
        if g + 1 < n_groups:
            pltpu.async_copy(src_hbm.at[wid, g + 1], si.at[1 - s], isem_s)
            pltpu.async_copy(dst_hbm.at[wid, g + 1], di.at[1 - s], isem_d)

        def step(k, b, s=s):
            pltpu.make_async_copy(x_hbm.at[si.at[s, k]], rows_v.at[b], gsems[b]).wait()
            pltpu.sync_copy(rows_v.at[b], acc.at[di.at[s, k]], add=True)

        def body(it, _, s=s):
            kk = it * NBUF
            for b in range(NBUF):
                step(kk + b, b)
                pltpu.async_copy(x_hbm.at[si.at[s, kk + b + NBUF]], rows_v.at[b],
                                 gsems[b])
            return 0

        lax.fori_loop(0, (IDXB - NBUF) // NBUF, body, 0)
        for b in range(NBUF):
            step(IDXB - NBUF + b, b)
        if g + 1 < n_groups:
            pltpu.make_async_copy(src_hbm.at[wid, g + 1], si.at[1 - s], isem_s).wait()
            pltpu.make_async_copy(dst_hbm.at[wid, g + 1], di.at[1 - s], isem_d).wait()
    plsc.subcore_barrier()
    pltpu.sync_copy(acc.at[pl.ds(base, rows_per_tile)],
                    out_hbm.at[cid, pl.ds(base, rows_per_tile)])


@functools.lru_cache(maxsize=None)
def _make_prop(n_groups):
    mesh = plsc.VectorSubcoreMesh(core_axis_name="c", subcore_axis_name="s")
    return pl.kernel(
        functools.partial(_prop_body, n_groups),
        out_type=jax.ShapeDtypeStruct((NC, NPAD, D_IN), jnp.float32),
        mesh=mesh,
        scratch_types=[
            pltpu.VMEM((2, IDXB, CH), jnp.int32),
            pltpu.VMEM((2, IDXB, CH), jnp.int32),
            pltpu.VMEM((NBUF, CH, D_IN), jnp.float32),
            pltpu.VMEM_SHARED((NPAD, D_IN), jnp.float32),
        ] + [pltpu.SemaphoreType.DMA] * (NBUF + 2),
    )


BR = 2048
GRID = NPAD // BR


def _scale_body(deg_ref, x_ref, isd_ref, xs_ref):
    d = deg_ref[...]
    s = lax.rsqrt(jnp.maximum(d[:, 0:1] + d[:, 1:2], 1.0))
    isd_ref[...] = s
    xs_ref[...] = x_ref[...] * s


_scale = pl.pallas_call(
    _scale_body,
    grid=(GRID,),
    in_specs=[pl.BlockSpec((BR, NC), lambda i: (i, 0)),
              pl.BlockSpec((BR, D_IN), lambda i: (i, 0))],
    out_specs=[pl.BlockSpec((BR, 1), lambda i: (i, 0)),
               pl.BlockSpec((BR, D_IN), lambda i: (i, 0))],
    out_shape=[jax.ShapeDtypeStruct((NPAD, 1), jnp.float32),
               jax.ShapeDtypeStruct((NPAD, D_IN), jnp.float32)],
)


def _mid_body(p_ref, isd_ref, w1_ref, b1_ref, w2_ref, out_ref):
    p = (p_ref[0] + p_ref[1]) * isd_ref[...]
    h1 = jnp.dot(p, w1_ref[...], preferred_element_type=jnp.float32)
    h1 = jnp.maximum(h1 + b1_ref[...], 0.0)
    g = jnp.dot(h1, w2_ref[...], preferred_element_type=jnp.float32)
    out_ref[...] = g * isd_ref[...]


_mid = pl.pallas_call(
    _mid_body,
    grid=(GRID,),
    in_specs=[pl.BlockSpec((NC, BR, D_IN), lambda i: (0, i, 0)),
              pl.BlockSpec((BR, 1), lambda i: (i, 0)),
              pl.BlockSpec((D_IN, D_HID), lambda i: (0, 0)),
              pl.BlockSpec((1, D_HID), lambda i: (0, 0)),
              pl.BlockSpec((D_HID, D_IN), lambda i: (0, 0))],
    out_specs=pl.BlockSpec((BR, D_IN), lambda i: (i, 0)),
    out_shape=jax.ShapeDtypeStruct((NPAD, D_IN), jnp.float32),
)


def _fin_body(q_ref, isd_ref, b2_ref, wc_ref, bc_ref, out_ref, acc_ref):
    i = pl.program_id(0)

    @pl.when(i == 0)
    def _():
        acc_ref[...] = jnp.zeros_like(acc_ref)

    q = (q_ref[0] + q_ref[1]) * isd_ref[...]
    h2 = jnp.maximum(q + b2_ref[...], 0.0)
    rows = i * BR + lax.broadcasted_iota(jnp.int32, (BR, 1), 0)
    h2 = jnp.where(rows < N_NODES, h2, 0.0)
    acc_ref[...] += jnp.sum(h2, axis=0, keepdims=True)

    @pl.when(i == GRID - 1)
    def _():
        g = acc_ref[...] * (1.0 / N_NODES)
        out_ref[...] = jnp.dot(g, wc_ref[...],
                               preferred_element_type=jnp.float32) + bc_ref[...]


_fin = pl.pallas_call(
    _fin_body,
    grid=(GRID,),
    in_specs=[pl.BlockSpec((NC, BR, D_IN), lambda i: (0, i, 0)),
              pl.BlockSpec((BR, 1), lambda i: (i, 0)),
              pl.BlockSpec((1, D_IN), lambda i: (0, 0)),
              pl.BlockSpec((D_IN, N_CLS), lambda i: (0, 0)),
              pl.BlockSpec((1, N_CLS), lambda i: (0, 0))],
    out_specs=pl.BlockSpec((1, N_CLS), lambda i: (0, 0)),
    out_shape=jax.ShapeDtypeStruct((1, N_CLS), jnp.float32),
    scratch_shapes=[pltpu.VMEM((1, D_IN), jnp.float32)],
)


def kernel(x, edge_index, W1, b1, W2, b2, Wc, bc):
    src = edge_index[0]
    dst = edge_index[1]
    e = src.shape[0]
    n_groups = -(-e // (NW * CH * IDXB))
    e_pad = NW * CH * IDXB * n_groups
    fill = N_NODES + (jnp.arange(e_pad - e, dtype=jnp.int32) % (NPAD - N_NODES))
    srcr = jnp.concatenate([src, fill]).reshape(NW, n_groups, IDXB, CH)
    dstr = jnp.concatenate([dst, fill]).reshape(NW, n_groups, IDXB, CH)
    x_pad = jnp.zeros((NPAD, D_IN), jnp.float32).at[:N_NODES].set(x)

    deg2 = _make_deg(n_groups)(dstr)
    isd, xs = _scale(deg2.T, x_pad)
    p = _make_prop(n_groups)(xs, srcr, dstr)
    g = _mid(p, isd, W1, b1.reshape(1, -1), W2)
    q = _make_prop(n_groups)(g, srcr, dstr)
    logits = _fin(q, isd, b2.reshape(1, -1), Wc, bc.reshape(1, -1))
    return logits.reshape(N_CLS)


# speedup vs baseline: 3.1385x; 1.0003x over previous
import functools

import jax
import jax.numpy as jnp
from jax import lax
from jax.experimental import pallas as pl
from jax.experimental.pallas import tpu as pltpu
from jax.experimental.pallas import tpu_sc as plsc

N_NODES = 10000
D_IN = 128
D_HID = 256
N_CLS = 4

NC = 2
NS = 16
NW = NC * NS
CH = 64
NBUF = 4
IDXB = 32
LANES = 16
NPAD = 10240


def _fill1d(ref, value):
    v = jnp.full((LANES,), value, jnp.float32)
    for j in range(CH // LANES):
        ref[pl.ds(j * LANES, LANES)] = v


def _zero2d(ref):
    z = jnp.zeros((LANES,), jnp.float32)

    def body(r, _):
        for j in range(D_IN // LANES):
            ref[r, pl.ds(j * LANES, LANES)] = z
        return 0

    lax.fori_loop(0, CH, body, 0)


def _deg_body(n_groups, dst_hbm, out_hbm, dst_v, ones_v, zero_v, acc):
    cid = lax.axis_index("c")
    sid = lax.axis_index("s")
    wid = sid * NC + cid
    pltpu.sync_copy(dst_hbm.at[wid], dst_v)
    _fill1d(ones_v, 1.0)
    _fill1d(zero_v, 0.0)
    words = NPAD // NS
    base = sid * words
    for j in range(words // CH):
        pltpu.sync_copy(zero_v, acc.at[pl.ds(base + j * CH, CH)])
    plsc.subcore_barrier()

    def body(k, _):
        pltpu.sync_copy(ones_v, acc.at[dst_v.at[k // IDXB, k % IDXB]], add=True)
        return 0

    lax.fori_loop(0, n_groups * IDXB, body, 0)
    plsc.subcore_barrier()
    pltpu.sync_copy(acc.at[pl.ds(base, words)], out_hbm.at[cid, pl.ds(base, words)])


@functools.lru_cache(maxsize=None)
def _make_deg(n_groups):
    mesh = plsc.VectorSubcoreMesh(core_axis_name="c", subcore_axis_name="s")
    return pl.kernel(
        functools.partial(_deg_body, n_groups),
        out_type=jax.ShapeDtypeStruct((NC, NPAD), jnp.float32),
        mesh=mesh,
        scratch_types=[
            pltpu.VMEM((n_groups, IDXB, CH), jnp.int32),
            pltpu.VMEM((CH,), jnp.float32),
            pltpu.VMEM((CH,), jnp.float32),
            pltpu.VMEM_SHARED((NPAD,), jnp.float32),
        ],
    )


def _prop_body(n_groups, x_hbm, src_hbm, dst_hbm, out_hbm,
               si, di, rows_v, acc, *sems_all):
    gsems = sems_all[:NBUF]
    isem_s, isem_d = sems_all[NBUF:]
    cid = lax.axis_index("c")
    sid = lax.axis_index("s")
    wid = sid * NC + cid
    pltpu.async_copy(src_hbm.at[wid, 0], si.at[0], isem_s)
    pltpu.async_copy(dst_hbm.at[wid, 0], di.at[0], isem_d)
    _zero2d(rows_v.at[0])
    rows_per_tile = NPAD // NS
    base = sid * rows_per_tile
    for j in range(rows_per_tile // CH):
        pltpu.sync_copy(rows_v.at[0], acc.at[pl.ds(base + j * CH, CH)])
    plsc.subcore_barrier()
    pltpu.make_async_copy(src_hbm.at[wid, 0], si.at[0], isem_s).wait()
    pltpu.make_async_copy(dst_hbm.at[wid, 0], di.at[0], isem_d).wait()

    for g in range(n_groups):
        s = g % 2

        for b in range(NBUF):
            pltpu.async_copy(x_hbm.at[si.at[s, b]], rows_v.at[b], gsems[b])
        if g + 1 < n_groups:
            pltpu.async_copy(src_hbm.at[wid, g + 1], si.at[1 - s], isem_s)
            pltpu.async_copy(dst_hbm.at[wid, g + 1], di.at[1 - s], isem_d)

        def step(k, b, s=s):
            pltpu.make_async_copy(x_hbm.at[si.at[s, k]], rows_v.at[b], gsems[b]).wait()
            pltpu.sync_copy(rows_v.at[b], acc.at[di.at[s, k]], add=True)

        def body(it, _, s=s):
            kk = it * NBUF
            for b in range(NBUF):
                step(kk + b, b)
                pltpu.async_copy(x_hbm.at[si.at[s, kk + b + NBUF]], rows_v.at[b],
                                 gsems[b])
            return 0

        lax.fori_loop(0, (IDXB - NBUF) // NBUF, body, 0)
        for b in range(NBUF):
            step(IDXB - NBUF + b, b)
        if g + 1 < n_groups:
            pltpu.make_async_copy(src_hbm.at[wid, g + 1], si.at[1 - s], isem_s).wait()
            pltpu.make_async_copy(dst_hbm.at[wid, g + 1], di.at[1 - s], isem_d).wait()
    plsc.subcore_barrier()
    pltpu.sync_copy(acc.at[pl.ds(base, rows_per_tile)],
                    out_hbm.at[cid, pl.ds(base, rows_per_tile)])


@functools.lru_cache(maxsize=None)
def _make_prop(n_groups):
    mesh = plsc.VectorSubcoreMesh(core_axis_name="c", subcore_axis_name="s")
    return pl.kernel(
        functools.partial(_prop_body, n_groups),
        out_type=jax.ShapeDtypeStruct((NC, NPAD, D_IN), jnp.float32),
        mesh=mesh,
        scratch_types=[
            pltpu.VMEM((2, IDXB, CH), jnp.int32),
            pltpu.VMEM((2, IDXB, CH), jnp.int32),
            pltpu.VMEM((NBUF, CH, D_IN), jnp.float32),
            pltpu.VMEM_SHARED((NPAD, D_IN), jnp.float32),
        ] + [pltpu.SemaphoreType.DMA] * (NBUF + 2),
    )


BR = 2000
GRID = N_NODES // BR


def _scale_body(deg_ref, x_ref, isd_ref, xs_ref):
    d = deg_ref[...]
    s = lax.rsqrt(jnp.maximum(d[:, 0:1] + d[:, 1:2], 1.0))
    isd_ref[...] = s
    xs_ref[...] = x_ref[...] * s


_scale = pl.pallas_call(
    _scale_body,
    grid=(GRID,),
    in_specs=[pl.BlockSpec((BR, NC), lambda i: (i, 0)),
              pl.BlockSpec((BR, D_IN), lambda i: (i, 0))],
    out_specs=[pl.BlockSpec((BR, 1), lambda i: (i, 0)),
               pl.BlockSpec((BR, D_IN), lambda i: (i, 0))],
    out_shape=[jax.ShapeDtypeStruct((N_NODES, 1), jnp.float32),
               jax.ShapeDtypeStruct((N_NODES, D_IN), jnp.float32)],
)


def _mid_body(p_ref, isd_ref, w1_ref, b1_ref, w2_ref, out_ref):
    p = (p_ref[0] + p_ref[1]) * isd_ref[...]
    h1 = jnp.dot(p, w1_ref[...], preferred_element_type=jnp.float32)
    h1 = jnp.maximum(h1 + b1_ref[...], 0.0)
    g = jnp.dot(h1, w2_ref[...], preferred_element_type=jnp.float32)
    out_ref[...] = g * isd_ref[...]


_mid = pl.pallas_call(
    _mid_body,
    grid=(GRID,),
    in_specs=[pl.BlockSpec((NC, BR, D_IN), lambda i: (0, i, 0)),
              pl.BlockSpec((BR, 1), lambda i: (i, 0)),
              pl.BlockSpec((D_IN, D_HID), lambda i: (0, 0)),
              pl.BlockSpec((1, D_HID), lambda i: (0, 0)),
              pl.BlockSpec((D_HID, D_IN), lambda i: (0, 0))],
    out_specs=pl.BlockSpec((BR, D_IN), lambda i: (i, 0)),
    out_shape=jax.ShapeDtypeStruct((N_NODES, D_IN), jnp.float32),
)


def _fin_body(q_ref, isd_ref, b2_ref, wc_ref, bc_ref, out_ref, acc_ref):
    i = pl.program_id(0)

    @pl.when(i == 0)
    def _():
        acc_ref[...] = jnp.zeros_like(acc_ref)

    q = (q_ref[0] + q_ref[1]) * isd_ref[...]
    h2 = jnp.maximum(q + b2_ref[...], 0.0)
    acc_ref[...] += jnp.sum(h2, axis=0, keepdims=True)

    @pl.when(i == GRID - 1)
    def _():
        g = acc_ref[...] * (1.0 / N_NODES)
        out_ref[...] = jnp.dot(g, wc_ref[...],
                               preferred_element_type=jnp.float32) + bc_ref[...]


_fin = pl.pallas_call(
    _fin_body,
    grid=(GRID,),
    in_specs=[pl.BlockSpec((NC, BR, D_IN), lambda i: (0, i, 0)),
              pl.BlockSpec((BR, 1), lambda i: (i, 0)),
              pl.BlockSpec((1, D_IN), lambda i: (0, 0)),
              pl.BlockSpec((D_IN, N_CLS), lambda i: (0, 0)),
              pl.BlockSpec((1, N_CLS), lambda i: (0, 0))],
    out_specs=pl.BlockSpec((1, N_CLS), lambda i: (0, 0)),
    out_shape=jax.ShapeDtypeStruct((1, N_CLS), jnp.float32),
    scratch_shapes=[pltpu.VMEM((1, D_IN), jnp.float32)],
)


def kernel(x, edge_index, W1, b1, W2, b2, Wc, bc):
    src = edge_index[0]
    dst = edge_index[1]
    e = src.shape[0]
    n_groups = -(-e // (NW * CH * IDXB))
    e_pad = NW * CH * IDXB * n_groups
    pad_n = e_pad - e
    fill_src = jnp.arange(pad_n, dtype=jnp.int32) % N_NODES
    fill_dst = N_NODES + (jnp.arange(pad_n, dtype=jnp.int32) % (NPAD - N_NODES))
    srcr = jnp.concatenate([src, fill_src]).reshape(NW, n_groups, IDXB, CH)
    dstr = jnp.concatenate([dst, fill_dst]).reshape(NW, n_groups, IDXB, CH)

    deg2 = _make_deg(n_groups)(dstr)
    isd, xs = _scale(deg2.T, x)
    p = _make_prop(n_groups)(xs, srcr, dstr)
    g = _mid(p, isd, W1, b1.reshape(1, -1), W2)
    q = _make_prop(n_groups)(g, srcr, dstr)
    logits = _fin(q, isd, b2.reshape(1, -1), Wc, bc.reshape(1, -1))
    return logits.reshape(N_CLS)


# speedup vs baseline: 3.2234x; 1.0271x over previous
import functools

import jax
import jax.numpy as jnp
from jax import lax
from jax.experimental import pallas as pl
from jax.experimental.pallas import tpu as pltpu
from jax.experimental.pallas import tpu_sc as plsc

N_NODES = 10000
D_IN = 128
D_HID = 256
N_CLS = 4

NC = 2
NS = 16
NW = NC * NS
CH = 64
NBUF = 4
IDXB = 32
LANES = 16
NPAD = 10240


def _fill1d(ref, value):
    v = jnp.full((LANES,), value, jnp.float32)
    for j in range(CH // LANES):
        ref[pl.ds(j * LANES, LANES)] = v


def _zero2d(ref):
    z = jnp.zeros((LANES,), jnp.float32)

    def body(r, _):
        for j in range(D_IN // LANES):
            ref[r, pl.ds(j * LANES, LANES)] = z
        return 0

    lax.fori_loop(0, CH, body, 0)


def _deg_body(n_groups, dst_hbm, out_hbm, dst_v, ones_v, zero_v, acc, sa, sb, zsem):
    cid = lax.axis_index("c")
    sid = lax.axis_index("s")
    wid = sid * NC + cid
    pltpu.sync_copy(dst_hbm.at[wid], dst_v)
    _fill1d(ones_v, 1.0)
    _fill1d(zero_v, 0.0)
    words = NPAD // NS
    base = sid * words
    for j in range(words // CH):
        pltpu.async_copy(zero_v, acc.at[pl.ds(base + j * CH, CH)], zsem)
    for j in range(words // CH):
        pltpu.make_async_copy(zero_v, acc.at[pl.ds(base + j * CH, CH)], zsem).wait()
    plsc.subcore_barrier()

    def dchunk(k):
        return acc.at[dst_v.at[k // IDXB, k % IDXB]]

    n_chunks = n_groups * IDXB
    pltpu.async_copy(ones_v, dchunk(0), sa, add=True)
    pltpu.async_copy(ones_v, dchunk(1), sb, add=True)

    def body(it, _):
        k2 = it * 2
        for b, sem in ((0, sa), (1, sb)):
            k = k2 + b
            pltpu.make_async_copy(ones_v, dchunk(k), sem).wait()
            pltpu.async_copy(ones_v, dchunk(k + 2), sem, add=True)
        return 0

    lax.fori_loop(0, (n_chunks - 2) // 2, body, 0)
    pltpu.make_async_copy(ones_v, dchunk(n_chunks - 2), sa).wait()
    pltpu.make_async_copy(ones_v, dchunk(n_chunks - 1), sb).wait()
    plsc.subcore_barrier()
    pltpu.sync_copy(acc.at[pl.ds(base, words)], out_hbm.at[cid, pl.ds(base, words)])


@functools.lru_cache(maxsize=None)
def _make_deg(n_groups):
    mesh = plsc.VectorSubcoreMesh(core_axis_name="c", subcore_axis_name="s")
    return pl.kernel(
        functools.partial(_deg_body, n_groups),
        out_type=jax.ShapeDtypeStruct((NC, NPAD), jnp.float32),
        mesh=mesh,
        scratch_types=[
            pltpu.VMEM((n_groups, IDXB, CH), jnp.int32),
            pltpu.VMEM((CH,), jnp.float32),
            pltpu.VMEM((CH,), jnp.float32),
            pltpu.VMEM_SHARED((NPAD,), jnp.float32),
            pltpu.SemaphoreType.DMA,
            pltpu.SemaphoreType.DMA,
            pltpu.SemaphoreType.DMA,
        ],
    )


def _prop_body(n_groups, x_hbm, src_hbm, dst_hbm, out_hbm,
               si, di, rows_v, acc, *sems_all):
    gsems = sems_all[:NBUF]
    isem_s, isem_d = sems_all[NBUF:]
    cid = lax.axis_index("c")
    sid = lax.axis_index("s")
    wid = sid * NC + cid
    pltpu.async_copy(src_hbm.at[wid, 0], si.at[0], isem_s)
    pltpu.async_copy(dst_hbm.at[wid, 0], di.at[0], isem_d)
    _zero2d(rows_v.at[0])
    rows_per_tile = NPAD // NS
    base = sid * rows_per_tile
    for j in range(rows_per_tile // CH):
        pltpu.async_copy(rows_v.at[0], acc.at[pl.ds(base + j * CH, CH)], gsems[0])
    for j in range(rows_per_tile // CH):
        pltpu.make_async_copy(rows_v.at[0], acc.at[pl.ds(base + j * CH, CH)],
                              gsems[0]).wait()
    plsc.subcore_barrier()
    pltpu.make_async_copy(src_hbm.at[wid, 0], si.at[0], isem_s).wait()
    pltpu.make_async_copy(dst_hbm.at[wid, 0], di.at[0], isem_d).wait()

    for g in range(n_groups):
        s = g % 2

        for b in range(NBUF):
            pltpu.async_copy(x_hbm.at[si.at[s, b]], rows_v.at[b], gsems[b])
        if g + 1 < n_groups:
            pltpu.async_copy(src_hbm.at[wid, g + 1], si.at[1 - s], isem_s)
            pltpu.async_copy(dst_hbm.at[wid, g + 1], di.at[1 - s], isem_d)

        def step(k, b, s=s):
            pltpu.make_async_copy(x_hbm.at[si.at[s, k]], rows_v.at[b], gsems[b]).wait()
            pltpu.sync_copy(rows_v.at[b], acc.at[di.at[s, k]], add=True)

        def body(it, _, s=s):
            kk = it * NBUF
            for b in range(NBUF):
                step(kk + b, b)
                pltpu.async_copy(x_hbm.at[si.at[s, kk + b + NBUF]], rows_v.at[b],
                                 gsems[b])
            return 0

        lax.fori_loop(0, (IDXB - NBUF) // NBUF, body, 0)
        for b in range(NBUF):
            step(IDXB - NBUF + b, b)
        if g + 1 < n_groups:
            pltpu.make_async_copy(src_hbm.at[wid, g + 1], si.at[1 - s], isem_s).wait()
            pltpu.make_async_copy(dst_hbm.at[wid, g + 1], di.at[1 - s], isem_d).wait()
    plsc.subcore_barrier()
    pltpu.sync_copy(acc.at[pl.ds(base, rows_per_tile)],
                    out_hbm.at[cid, pl.ds(base, rows_per_tile)])


@functools.lru_cache(maxsize=None)
def _make_prop(n_groups):
    mesh = plsc.VectorSubcoreMesh(core_axis_name="c", subcore_axis_name="s")
    return pl.kernel(
        functools.partial(_prop_body, n_groups),
        out_type=jax.ShapeDtypeStruct((NC, NPAD, D_IN), jnp.float32),
        mesh=mesh,
        scratch_types=[
            pltpu.VMEM((2, IDXB, CH), jnp.int32),
            pltpu.VMEM((2, IDXB, CH), jnp.int32),
            pltpu.VMEM((NBUF, CH, D_IN), jnp.float32),
            pltpu.VMEM_SHARED((NPAD, D_IN), jnp.float32),
        ] + [pltpu.SemaphoreType.DMA] * (NBUF + 2),
    )


BR = 2000
GRID = N_NODES // BR


def _scale_body(deg_ref, x_ref, isd_ref, xs_ref):
    d = deg_ref[...]
    s = lax.rsqrt(jnp.maximum(d[:, 0:1] + d[:, 1:2], 1.0))
    isd_ref[...] = s
    xs_ref[...] = x_ref[...] * s


_scale = pl.pallas_call(
    _scale_body,
    grid=(GRID,),
    in_specs=[pl.BlockSpec((BR, NC), lambda i: (i, 0)),
              pl.BlockSpec((BR, D_IN), lambda i: (i, 0))],
    out_specs=[pl.BlockSpec((BR, 1), lambda i: (i, 0)),
               pl.BlockSpec((BR, D_IN), lambda i: (i, 0))],
    out_shape=[jax.ShapeDtypeStruct((N_NODES, 1), jnp.float32),
               jax.ShapeDtypeStruct((N_NODES, D_IN), jnp.float32)],
)


def _mid_body(p_ref, isd_ref, w1_ref, b1_ref, w2_ref, out_ref):
    p = (p_ref[0] + p_ref[1]) * isd_ref[...]
    h1 = jnp.dot(p, w1_ref[...], preferred_element_type=jnp.float32)
    h1 = jnp.maximum(h1 + b1_ref[...], 0.0)
    g = jnp.dot(h1, w2_ref[...], preferred_element_type=jnp.float32)
    out_ref[...] = g * isd_ref[...]


_mid = pl.pallas_call(
    _mid_body,
    grid=(GRID,),
    in_specs=[pl.BlockSpec((NC, BR, D_IN), lambda i: (0, i, 0)),
              pl.BlockSpec((BR, 1), lambda i: (i, 0)),
              pl.BlockSpec((D_IN, D_HID), lambda i: (0, 0)),
              pl.BlockSpec((1, D_HID), lambda i: (0, 0)),
              pl.BlockSpec((D_HID, D_IN), lambda i: (0, 0))],
    out_specs=pl.BlockSpec((BR, D_IN), lambda i: (i, 0)),
    out_shape=jax.ShapeDtypeStruct((N_NODES, D_IN), jnp.float32),
)


def _fin_body(q_ref, isd_ref, b2_ref, wc_ref, bc_ref, out_ref, acc_ref):
    i = pl.program_id(0)

    @pl.when(i == 0)
    def _():
        acc_ref[...] = jnp.zeros_like(acc_ref)

    q = (q_ref[0] + q_ref[1]) * isd_ref[...]
    h2 = jnp.maximum(q + b2_ref[...], 0.0)
    acc_ref[...] += jnp.sum(h2, axis=0, keepdims=True)

    @pl.when(i == GRID - 1)
    def _():
        g = acc_ref[...] * (1.0 / N_NODES)
        out_ref[...] = jnp.dot(g, wc_ref[...],
                               preferred_element_type=jnp.float32) + bc_ref[...]


_fin = pl.pallas_call(
    _fin_body,
    grid=(GRID,),
    in_specs=[pl.BlockSpec((NC, BR, D_IN), lambda i: (0, i, 0)),
              pl.BlockSpec((BR, 1), lambda i: (i, 0)),
              pl.BlockSpec((1, D_IN), lambda i: (0, 0)),
              pl.BlockSpec((D_IN, N_CLS), lambda i: (0, 0)),
              pl.BlockSpec((1, N_CLS), lambda i: (0, 0))],
    out_specs=pl.BlockSpec((1, N_CLS), lambda i: (0, 0)),
    out_shape=jax.ShapeDtypeStruct((1, N_CLS), jnp.float32),
    scratch_shapes=[pltpu.VMEM((1, D_IN), jnp.float32)],
)


def kernel(x, edge_index, W1, b1, W2, b2, Wc, bc):
    src = edge_index[0]
    dst = edge_index[1]
    e = src.shape[0]
    n_groups = -(-e // (NW * CH * IDXB))
    e_pad = NW * CH * IDXB * n_groups
    pad_n = e_pad - e
    fill_src = jnp.arange(pad_n, dtype=jnp.int32) % N_NODES
    fill_dst = N_NODES + (jnp.arange(pad_n, dtype=jnp.int32) % (NPAD - N_NODES))
    srcr = jnp.concatenate([src, fill_src]).reshape(NW, n_groups, IDXB, CH)
    dstr = jnp.concatenate([dst, fill_dst]).reshape(NW, n_groups, IDXB, CH)

    deg2 = _make_deg(n_groups)(dstr)
    isd, xs = _scale(deg2.T, x)
    p = _make_prop(n_groups)(xs, srcr, dstr)
    g = _mid(p, isd, W1, b1.reshape(1, -1), W2)
    q = _make_prop(n_groups)(g, srcr, dstr)
    logits = _fin(q, isd, b2.reshape(1, -1), Wc, bc.reshape(1, -1))
    return logits.reshape(N_CLS)
